# baseline (speedup 1.0000x reference)
import functools

import jax
import jax.numpy as jnp
from jax.experimental import pallas as pl

_DI1 = 32
_DI2 = 32
_DO1 = 32
_DO2 = 32
_E = 64
_D = _DI1 * _DI2
_DOUT = _DO1 * _DO2
_T = 256


def _moe_block(x_ref, x3_ref, w_ref, ab_ref, scale_ref, bias_ref, out_ref):
    xb = x_ref[...]
    w = w_ref[...]

    logits = jax.lax.dot_general(
        xb, w, (((1,), (1,)), ((), ())),
        preferred_element_type=jnp.float32,
    )

    iota = jax.lax.broadcasted_iota(jnp.int32, logits.shape, 1)
    neg = jnp.float32(-1e30)

    m1 = jnp.max(logits, axis=1, keepdims=True)
    idx1 = jnp.min(jnp.where(logits == m1, iota, _E), axis=1, keepdims=True)
    masked = jnp.where(iota == idx1, neg, logits)
    m2 = jnp.max(masked, axis=1, keepdims=True)
    idx2 = jnp.min(jnp.where(masked == m2, iota, _E), axis=1, keepdims=True)

    p1 = 1.0 / (1.0 + jnp.exp(m2 - m1))
    p2 = 1.0 - p1

    oh1 = (iota == idx1).astype(jnp.float32)
    oh2 = (iota == idx2).astype(jnp.float32)

    tab = ab_ref[...]

    dn = (((1,), (0,)), ((), ()))
    g1 = jax.lax.dot_general(oh1, tab, dn, preferred_element_type=jnp.float32)
    g2 = jax.lax.dot_general(oh2, tab, dn, preferred_element_type=jnp.float32)

    x3 = x3_ref[...]

    def expert_apply(g):
        a3 = g[:, :_D].reshape(_T, _DO1, _DI1)
        bt3 = g[:, _D:].reshape(_T, _DI2, _DO2)
        u = jax.lax.dot_general(
            x3, bt3, (((2,), (1,)), ((0,), (0,))),
            preferred_element_type=jnp.float32)
        y = jax.lax.dot_general(
            a3, u, (((2,), (1,)), ((0,), (0,))),
            preferred_element_type=jnp.float32)
        return y

    y1 = expert_apply(g1)
    y2 = expert_apply(g2)

    out = y1 * p1[:, :, None] + y2 * p2[:, :, None]
    out_ref[...] = out * scale_ref[0, 0] + bias_ref[...]


@jax.jit
def kernel(x, W_router, A_experts, B_experts, scale, bias):
    orig_shape = x.shape
    x_flat = x.reshape(-1, _D)
    n_tok = x_flat.shape[0]

    a_tab = A_experts.reshape(_E, _DO1 * _DI1)
    bt_tab = B_experts.transpose(0, 2, 1).reshape(_E, _DI2 * _DO2)
    ab_tab = jnp.concatenate([a_tab, bt_tab], axis=1)
    scale2 = scale.reshape(1, 1)
    bias3 = bias.reshape(1, _DO1, _DO2)
    x_3d = x_flat.reshape(n_tok, _DI1, _DI2)

    grid = (n_tok // _T,)
    out = pl.pallas_call(
        _moe_block,
        grid=grid,
        in_specs=[
            pl.BlockSpec((_T, _D), lambda i: (i, 0)),
            pl.BlockSpec((_T, _DI1, _DI2), lambda i: (i, 0, 0)),
            pl.BlockSpec((_E, _D), lambda i: (0, 0)),
            pl.BlockSpec((_E, 2 * _D), lambda i: (0, 0)),
            pl.BlockSpec((1, 1), lambda i: (0, 0)),
            pl.BlockSpec((1, _DO1, _DO2), lambda i: (0, 0, 0)),
        ],
        out_specs=pl.BlockSpec((_T, _DO1, _DO2), lambda i: (i, 0, 0)),
        out_shape=jax.ShapeDtypeStruct((n_tok, _DO1, _DO2), jnp.float32),
    )(x_flat, x_3d, W_router, ab_tab, scale2, bias3)

    return out.reshape(orig_shape[0], orig_shape[1], _DOUT)


# speedup vs baseline: 1.3892x; 1.3892x over previous
import functools

import jax
import jax.numpy as jnp
from jax.experimental import pallas as pl

_DI1 = 32
_DI2 = 32
_DO1 = 32
_DO2 = 32
_E = 64
_D = _DI1 * _DI2
_DOUT = _DO1 * _DO2
_T = 256


def _moe_block(x_ref, w_ref, ab_ref, scale_ref, bias_ref, out_ref):
    xb = x_ref[...]
    w = w_ref[...]

    logits = jax.lax.dot_general(
        xb, w, (((1,), (1,)), ((), ())),
        preferred_element_type=jnp.float32,
    )

    iota = jax.lax.broadcasted_iota(jnp.int32, logits.shape, 1)
    neg = jnp.float32(-1e30)

    m1 = jnp.max(logits, axis=1, keepdims=True)
    idx1 = jnp.min(jnp.where(logits == m1, iota, _E), axis=1, keepdims=True)
    masked = jnp.where(iota == idx1, neg, logits)
    m2 = jnp.max(masked, axis=1, keepdims=True)
    idx2 = jnp.min(jnp.where(masked == m2, iota, _E), axis=1, keepdims=True)

    p1 = 1.0 / (1.0 + jnp.exp(m2 - m1))
    p2 = 1.0 - p1

    oh1 = (iota == idx1).astype(jnp.float32)
    oh2 = (iota == idx2).astype(jnp.float32)

    tab = ab_ref[...]

    dn = (((1,), (0,)), ((), ()))
    g1 = jax.lax.dot_general(oh1, tab, dn, preferred_element_type=jnp.float32)
    g2 = jax.lax.dot_general(oh2, tab, dn, preferred_element_type=jnp.float32)

    x3 = xb.reshape(_T, _DI1, _DI2)

    def expert_apply(g):
        a3 = g[:, :_D].reshape(_T, _DO1, _DI1)
        bt3 = g[:, _D:].reshape(_T, _DI2, _DO2)
        u = jax.lax.dot_general(
            x3, bt3, (((2,), (1,)), ((0,), (0,))),
            preferred_element_type=jnp.float32)
        y = jax.lax.dot_general(
            a3, u, (((2,), (1,)), ((0,), (0,))),
            preferred_element_type=jnp.float32)
        return y

    y1 = expert_apply(g1)
    y2 = expert_apply(g2)

    out = y1 * p1[:, :, None] + y2 * p2[:, :, None]
    out = out.reshape(_T, _DOUT)
    out_ref[...] = out * scale_ref[0, 0] + bias_ref[...]


@jax.jit
def kernel(x, W_router, A_experts, B_experts, scale, bias):
    orig_shape = x.shape
    x_flat = x.reshape(-1, _D)
    n_tok = x_flat.shape[0]

    a_tab = A_experts.reshape(_E, _DO1 * _DI1)
    bt_tab = B_experts.transpose(0, 2, 1).reshape(_E, _DI2 * _DO2)
    ab_tab = jnp.concatenate([a_tab, bt_tab], axis=1)
    scale2 = scale.reshape(1, 1)
    bias2 = bias.reshape(1, _DOUT)

    grid = (n_tok // _T,)
    out = pl.pallas_call(
        _moe_block,
        grid=grid,
        in_specs=[
            pl.BlockSpec((_T, _D), lambda i: (i, 0)),
            pl.BlockSpec((_E, _D), lambda i: (0, 0)),
            pl.BlockSpec((_E, 2 * _D), lambda i: (0, 0)),
            pl.BlockSpec((1, 1), lambda i: (0, 0)),
            pl.BlockSpec((1, _DOUT), lambda i: (0, 0)),
        ],
        out_specs=pl.BlockSpec((_T, _DOUT), lambda i: (i, 0)),
        out_shape=jax.ShapeDtypeStruct((n_tok, _DOUT), jnp.float32),
    )(x_flat, W_router, ab_tab, scale2, bias2)

    return out.reshape(orig_shape[0], orig_shape[1], _DOUT)


# speedup vs baseline: 1.5649x; 1.1265x over previous
import functools

import jax
import jax.numpy as jnp
from jax.experimental import pallas as pl

_DI1 = 32
_DI2 = 32
_DO1 = 32
_DO2 = 32
_E = 64
_D = _DI1 * _DI2
_DOUT = _DO1 * _DO2
_T = 512


def _moe_block(x_ref, w_ref, ab_ref, scale_ref, bias_ref, out_ref):
    xb = x_ref[...]
    w = w_ref[...]

    logits = jax.lax.dot_general(
        xb, w, (((1,), (1,)), ((), ())),
        preferred_element_type=jnp.float32,
    )

    iota = jax.lax.broadcasted_iota(jnp.int32, logits.shape, 1)
    neg = jnp.float32(-1e30)

    m1 = jnp.max(logits, axis=1, keepdims=True)
    idx1 = jnp.min(jnp.where(logits == m1, iota, _E), axis=1, keepdims=True)
    masked = jnp.where(iota == idx1, neg, logits)
    m2 = jnp.max(masked, axis=1, keepdims=True)
    idx2 = jnp.min(jnp.where(masked == m2, iota, _E), axis=1, keepdims=True)

    p1 = 1.0 / (1.0 + jnp.exp(m2 - m1))
    p2 = 1.0 - p1

    oh1 = (iota == idx1).astype(jnp.float32)
    oh2 = (iota == idx2).astype(jnp.float32)

    tab = ab_ref[...]

    dn = (((1,), (0,)), ((), ()))
    g1 = jax.lax.dot_general(oh1, tab, dn, preferred_element_type=jnp.float32)
    g2 = jax.lax.dot_general(oh2, tab, dn, preferred_element_type=jnp.float32)

    x3 = xb.reshape(_T, _DI1, _DI2)

    def expert_apply(g):
        a3 = g[:, :_D].reshape(_T, _DO1, _DI1)
        b3 = g[:, _D:].reshape(_T, _DO2, _DI2)
        u = jax.lax.dot_general(
            x3, b3, (((2,), (2,)), ((0,), (0,))),
            preferred_element_type=jnp.float32)
        y = jax.lax.dot_general(
            a3, u, (((2,), (1,)), ((0,), (0,))),
            preferred_element_type=jnp.float32)
        return y

    y1 = expert_apply(g1)
    y2 = expert_apply(g2)

    out = y1 * p1[:, :, None] + y2 * p2[:, :, None]
    out = out.reshape(_T, _DOUT)
    out_ref[...] = out * scale_ref[0, 0] + bias_ref[...]


@jax.jit
def kernel(x, W_router, A_experts, B_experts, scale, bias):
    orig_shape = x.shape
    x_flat = x.reshape(-1, _D)
    n_tok = x_flat.shape[0]

    a_tab = A_experts.reshape(_E, _DO1 * _DI1)
    b_tab = B_experts.reshape(_E, _DO2 * _DI2)
    ab_tab = jnp.concatenate([a_tab, b_tab], axis=1)
    scale2 = scale.reshape(1, 1)
    bias2 = bias.reshape(1, _DOUT)

    grid = (n_tok // _T,)
    out = pl.pallas_call(
        _moe_block,
        grid=grid,
        in_specs=[
            pl.BlockSpec((_T, _D), lambda i: (i, 0)),
            pl.BlockSpec((_E, _D), lambda i: (0, 0)),
            pl.BlockSpec((_E, 2 * _D), lambda i: (0, 0)),
            pl.BlockSpec((1, 1), lambda i: (0, 0)),
            pl.BlockSpec((1, _DOUT), lambda i: (0, 0)),
        ],
        out_specs=pl.BlockSpec((_T, _DOUT), lambda i: (i, 0)),
        out_shape=jax.ShapeDtypeStruct((n_tok, _DOUT), jnp.float32),
    )(x_flat, W_router, ab_tab, scale2, bias2)

    return out.reshape(orig_shape[0], orig_shape[1], _DOUT)


# speedup vs baseline: 1.5721x; 1.0046x over previous
import functools

import jax
import jax.numpy as jnp
from jax.experimental import pallas as pl

_DI1 = 32
_DI2 = 32
_DO1 = 32
_DO2 = 32
_E = 64
_D = _DI1 * _DI2
_DOUT = _DO1 * _DO2
_T = 512


def _moe_block(x_ref, w_ref, ab_ref, scale_ref, bias_ref, out_ref):
    xb = x_ref[...]
    w = w_ref[...]

    logits = jax.lax.dot_general(
        xb, w, (((1,), (1,)), ((), ())),
        preferred_element_type=jnp.float32,
    )

    iota = jax.lax.broadcasted_iota(jnp.int32, logits.shape, 1)
    neg = jnp.float32(-1e30)

    m1 = jnp.max(logits, axis=1, keepdims=True)
    idx1 = jnp.min(jnp.where(logits == m1, iota, _E), axis=1, keepdims=True)
    masked = jnp.where(iota == idx1, neg, logits)
    m2 = jnp.max(masked, axis=1, keepdims=True)
    idx2 = jnp.min(jnp.where(masked == m2, iota, _E), axis=1, keepdims=True)

    p1 = 1.0 / (1.0 + jnp.exp(m2 - m1))
    p2 = 1.0 - p1

    oh1 = (iota == idx1).astype(jnp.float32)
    oh2 = (iota == idx2).astype(jnp.float32)

    tab = ab_ref[...]

    oh = jnp.concatenate([oh1, oh2], axis=0)
    dn = (((1,), (0,)), ((), ()))
    g = jax.lax.dot_general(oh, tab, dn, preferred_element_type=jnp.float32)

    x3 = xb.reshape(_T, _DI1, _DI2)
    x3c = jnp.concatenate([x3, x3], axis=0)

    a3 = g[:, :_D].reshape(2 * _T, _DO1, _DI1)
    b3 = g[:, _D:].reshape(2 * _T, _DO2, _DI2)
    u = jax.lax.dot_general(
        x3c, b3, (((2,), (2,)), ((0,), (0,))),
        preferred_element_type=jnp.float32)
    y = jax.lax.dot_general(
        a3, u, (((2,), (1,)), ((0,), (0,))),
        preferred_element_type=jnp.float32)

    y1 = y[:_T]
    y2 = y[_T:]

    out = y1 * p1[:, :, None] + y2 * p2[:, :, None]
    out = out.reshape(_T, _DOUT)
    out_ref[...] = out * scale_ref[0, 0] + bias_ref[...]


@jax.jit
def kernel(x, W_router, A_experts, B_experts, scale, bias):
    orig_shape = x.shape
    x_flat = x.reshape(-1, _D)
    n_tok = x_flat.shape[0]

    a_tab = A_experts.reshape(_E, _DO1 * _DI1)
    b_tab = B_experts.reshape(_E, _DO2 * _DI2)
    ab_tab = jnp.concatenate([a_tab, b_tab], axis=1)
    scale2 = scale.reshape(1, 1)
    bias2 = bias.reshape(1, _DOUT)

    grid = (n_tok // _T,)
    out = pl.pallas_call(
        _moe_block,
        grid=grid,
        in_specs=[
            pl.BlockSpec((_T, _D), lambda i: (i, 0)),
            pl.BlockSpec((_E, _D), lambda i: (0, 0)),
            pl.BlockSpec((_E, 2 * _D), lambda i: (0, 0)),
            pl.BlockSpec((1, 1), lambda i: (0, 0)),
            pl.BlockSpec((1, _DOUT), lambda i: (0, 0)),
        ],
        out_specs=pl.BlockSpec((_T, _DOUT), lambda i: (i, 0)),
        out_shape=jax.ShapeDtypeStruct((n_tok, _DOUT), jnp.float32),
    )(x_flat, W_router, ab_tab, scale2, bias2)

    return out.reshape(orig_shape[0], orig_shape[1], _DOUT)
